# baseline (speedup 1.0000x reference)
import jax
import jax.numpy as jnp
from jax import lax
from jax.experimental import pallas as pl
from jax.experimental.pallas import tpu as pltpu
from jax.experimental.pallas import tpu_sc as plsc

VOCAB = 400000
EMBED_DIM = 50
DP = 64
BATCH = 4096
SEQ = 200

_INFO = plsc.get_sparse_core_info()
_NC = _INFO.num_cores
_NS = _INFO.num_subcores
_NW = _NC * _NS

_B = BATCH * SEQ
_PER_W = _B // _NW
_G = 128
_SUB = 4
_CHUNK = _G * _SUB
_STEPS = _PER_W // _CHUNK
_GROUPS = _CHUNK // 8
_XROWS = _PER_W // _G


def _pad_body(t_ref, o_ref):
    o_ref[...] = jnp.pad(t_ref[...], ((0, 0), (0, DP - EMBED_DIM)))


def _gather_body(x_hbm, tab_hbm, out_hbm, idx0, idx1, raw0, raw1, pk0, pk1,
                 si0, si1, sg0, sg1, ss0, ss1):
    wid = lax.axis_index("s") * _NC + lax.axis_index("c")
    xbase = wid * _XROWS
    obase = wid * _PER_W * EMBED_DIM

    idxb, rawb, pkb = (idx0, idx1), (raw0, raw1), (pk0, pk1)
    sib, sgb, ssb = (si0, si1), (sg0, sg1), (ss0, ss1)

    lane = lax.iota(jnp.int32, 16)
    jvs, cvs = [], []
    for g in range(25):
        v = lane + (16 * g)
        j = (v * 5243) >> 18
        jvs.append(j)
        cvs.append(v - j * 50)

    def fire_idx(i, b):
        pltpu.async_copy(x_hbm.at[pl.ds(xbase + i * _SUB, _SUB)],
                         idxb[b], sib[b])

    def fire_gathers(i, b):
        del i
        for j in range(_SUB):
            pltpu.async_copy(tab_hbm.at[idxb[b].at[j]],
                             rawb[b].at[pl.ds(j * _G, _G)], sgb[b])

    def drain(src, dst, sem):
        pltpu.make_async_copy(src, dst, sem).wait()

    def store(i, b):
        pltpu.async_copy(pkb[b],
                         out_hbm.at[pl.ds(obase + i * _CHUNK * EMBED_DIM,
                                          _CHUNK * EMBED_DIM)],
                         ssb[b])

    def compact(b):
        raw, pk = rawb[b], pkb[b]

        def group(m, c):
            rbase = m * 8
            pbase = m * 400
            for g in range(25):
                val = plsc.load_gather(raw, [jvs[g] + rbase, cvs[g]])
                pk[pl.ds(pbase + 16 * g, 16)] = val
            return c

        lax.fori_loop(0, _GROUPS, group, 0)

    pltpu.sync_copy(x_hbm.at[pl.ds(xbase, _SUB)], idx0)
    fire_idx(1, 1)
    fire_gathers(0, 0)

    def step(it, c):
        for b in (0, 1):
            i = 2 * it + b
            drain(tab_hbm.at[pl.ds(0, _CHUNK)], rawb[b], sgb[b])

            @pl.when(it < (_STEPS - 2 - b + 1) // 2)
            def _():
                fire_idx(i + 2, b)

            @pl.when(jnp.logical_or(b == 0, it < (_STEPS - 2) // 2))
            def _():
                drain(x_hbm.at[pl.ds(0, _SUB)], idxb[1 - b], sib[1 - b])
                fire_gathers(i + 1, 1 - b)

            @pl.when(it > 0)
            def _():
                drain(pkb[b],
                      out_hbm.at[pl.ds(0, _CHUNK * EMBED_DIM)], ssb[b])

            compact(b)
            store(i, b)
        return c

    lax.fori_loop(0, _STEPS // 2, step, 0)

    drain(pk0, out_hbm.at[pl.ds(0, _CHUNK * EMBED_DIM)], ss0)
    drain(pk1, out_hbm.at[pl.ds(0, _CHUNK * EMBED_DIM)], ss1)


def kernel(x, emb_table):
    idx = x.reshape(_B // _G, _G).astype(jnp.int32)

    pad = pl.pallas_call(
        _pad_body,
        grid=(VOCAB // 2000,),
        in_specs=[pl.BlockSpec((2000, EMBED_DIM), lambda i: (i, 0))],
        out_specs=pl.BlockSpec((2000, DP), lambda i: (i, 0)),
        out_shape=jax.ShapeDtypeStruct((VOCAB, DP), jnp.float32),
    )
    tab64 = pad(emb_table)

    gather = pl.kernel(
        _gather_body,
        out_type=jax.ShapeDtypeStruct((_B * EMBED_DIM,), jnp.float32),
        mesh=plsc.VectorSubcoreMesh(core_axis_name="c", subcore_axis_name="s"),
        scratch_types=[
            pltpu.VMEM((_SUB, _G), jnp.int32),
            pltpu.VMEM((_SUB, _G), jnp.int32),
            pltpu.VMEM((_CHUNK, DP), jnp.float32),
            pltpu.VMEM((_CHUNK, DP), jnp.float32),
            pltpu.VMEM((_CHUNK * EMBED_DIM,), jnp.float32),
            pltpu.VMEM((_CHUNK * EMBED_DIM,), jnp.float32),
            pltpu.SemaphoreType.DMA,
            pltpu.SemaphoreType.DMA,
            pltpu.SemaphoreType.DMA,
            pltpu.SemaphoreType.DMA,
            pltpu.SemaphoreType.DMA,
            pltpu.SemaphoreType.DMA,
        ],
        compiler_params=pltpu.CompilerParams(use_tc_tiling_on_sc=False,
                                             needs_layout_passes=False),
    )
    out = gather(idx, tab64)
    return out.reshape(BATCH, SEQ, EMBED_DIM)


# speedup vs baseline: 1.0827x; 1.0827x over previous
import jax
import jax.numpy as jnp
from jax import lax
from jax.experimental import pallas as pl
from jax.experimental.pallas import tpu as pltpu
from jax.experimental.pallas import tpu_sc as plsc

VOCAB = 400000
EMBED_DIM = 50
DP = 64
BATCH = 4096
SEQ = 200

_INFO = plsc.get_sparse_core_info()
_NC = _INFO.num_cores
_NS = _INFO.num_subcores
_NW = _NC * _NS

_B = BATCH * SEQ
_PER_W = _B // _NW
_G = 128
_SUB = 4
_CHUNK = _G * _SUB
_STEPS = _PER_W // _CHUNK
_GROUPS = _CHUNK // 8
_XROWS = _PER_W // _G


def _pad_body(t_ref, o_ref):
    o_ref[...] = jnp.pad(t_ref[...], ((0, 0), (0, DP - EMBED_DIM)))


def _gather_body(x_hbm, tab_hbm, out_hbm, idx0, idx1, raw0, raw1, pk0, pk1,
                 si0, si1, sg0, sg1, ss0, ss1):
    wid = lax.axis_index("s") * _NC + lax.axis_index("c")
    xbase = wid * _XROWS
    obase = wid * _PER_W * EMBED_DIM

    idxb, rawb, pkb = (idx0, idx1), (raw0, raw1), (pk0, pk1)
    sib, sgb, ssb = (si0, si1), (sg0, sg1), (ss0, ss1)

    lane = lax.iota(jnp.int32, 16)
    jvs, cvs = [], []
    for g in range(25):
        v = lane + (16 * g)
        j = (v * 5243) >> 18
        jvs.append(j)
        cvs.append(v - j * 50)

    def fire_idx(i, b):
        pltpu.async_copy(x_hbm.at[pl.ds(xbase + i * _SUB, _SUB)],
                         idxb[b], sib[b])

    def fire_gathers(i, b):
        del i
        for j in range(_SUB):
            pltpu.async_copy(tab_hbm.at[idxb[b].at[j]],
                             rawb[b].at[pl.ds(j * _G, _G)], sgb[b])

    def drain(src, dst, sem):
        pltpu.make_async_copy(src, dst, sem).wait()

    def store(i, b):
        pltpu.async_copy(pkb[b],
                         out_hbm.at[pl.ds(obase + i * _CHUNK * EMBED_DIM,
                                          _CHUNK * EMBED_DIM)],
                         ssb[b])

    def compact(b):
        raw, pk = rawb[b], pkb[b]

        def group(m, c):
            rbase = m * 8
            pbase = m * 400
            for g in range(25):
                val = plsc.load_gather(raw, [jvs[g] + rbase, cvs[g]])
                pk[pl.ds(pbase + 16 * g, 16)] = val
            return c

        lax.fori_loop(0, _GROUPS, group, 0)

    pltpu.sync_copy(x_hbm.at[pl.ds(xbase, _SUB)], idx0)
    fire_idx(1, 1)
    fire_gathers(0, 0)

    def step(it, c):
        for b in (0, 1):
            i = 2 * it + b
            drain(tab_hbm.at[pl.ds(0, _CHUNK)], rawb[b], sgb[b])

            @pl.when(it < (_STEPS - 2 - b + 1) // 2)
            def _():
                fire_idx(i + 2, b)

            @pl.when(jnp.logical_or(b == 0, it < (_STEPS - 2) // 2))
            def _():
                drain(x_hbm.at[pl.ds(0, _SUB)], idxb[1 - b], sib[1 - b])
                fire_gathers(i + 1, 1 - b)

            @pl.when(it > 0)
            def _():
                drain(pkb[b],
                      out_hbm.at[pl.ds(0, _CHUNK * EMBED_DIM)], ssb[b])

            compact(b)
            store(i, b)
        return c

    lax.fori_loop(0, _STEPS // 2, step, 0)

    drain(pk0, out_hbm.at[pl.ds(0, _CHUNK * EMBED_DIM)], ss0)
    drain(pk1, out_hbm.at[pl.ds(0, _CHUNK * EMBED_DIM)], ss1)


def kernel(x, emb_table):
    idx = x.reshape(_B // _G, _G).astype(jnp.int32)

    tab64 = jnp.pad(emb_table, ((0, 0), (0, DP - EMBED_DIM)))

    gather = pl.kernel(
        _gather_body,
        out_type=jax.ShapeDtypeStruct((_B * EMBED_DIM,), jnp.float32),
        mesh=plsc.VectorSubcoreMesh(core_axis_name="c", subcore_axis_name="s"),
        scratch_types=[
            pltpu.VMEM((_SUB, _G), jnp.int32),
            pltpu.VMEM((_SUB, _G), jnp.int32),
            pltpu.VMEM((_CHUNK, DP), jnp.float32),
            pltpu.VMEM((_CHUNK, DP), jnp.float32),
            pltpu.VMEM((_CHUNK * EMBED_DIM,), jnp.float32),
            pltpu.VMEM((_CHUNK * EMBED_DIM,), jnp.float32),
            pltpu.SemaphoreType.DMA,
            pltpu.SemaphoreType.DMA,
            pltpu.SemaphoreType.DMA,
            pltpu.SemaphoreType.DMA,
            pltpu.SemaphoreType.DMA,
            pltpu.SemaphoreType.DMA,
        ],
        compiler_params=pltpu.CompilerParams(use_tc_tiling_on_sc=False,
                                             needs_layout_passes=False),
    )
    out = gather(idx, tab64)
    return out.reshape(BATCH, SEQ, EMBED_DIM)


# speedup vs baseline: 2.1323x; 1.9695x over previous
import jax
import jax.numpy as jnp
from jax import lax
from jax.experimental import pallas as pl
from jax.experimental.pallas import tpu as pltpu
from jax.experimental.pallas import tpu_sc as plsc

VOCAB = 400000
EMBED_DIM = 50
DP = 128
BATCH = 4096
SEQ = 200

_INFO = plsc.get_sparse_core_info()
_NC = _INFO.num_cores
_NS = _INFO.num_subcores
_NW = _NC * _NS

_B = BATCH * SEQ
_PER_W = _B // _NW
_G = 128
_SUB = 1
_CHUNK = _G * _SUB
_STEPS = _PER_W // _CHUNK
_XROWS = _PER_W // _G

_WOFF = (0, 16, 32, 34)


def _gather_body(x_hbm, tab_hbm, out_hbm, idx_v, raw0, raw1, pk0, pk1,
                 sg0, sg1):
    wid = lax.axis_index("s") * _NC + lax.axis_index("c")
    obase = wid * _PER_W
    rawb, pkb, sgb = (raw0, raw1), (pk0, pk1), (sg0, sg1)

    def fire(i, b):
        for j in range(_SUB):
            pltpu.async_copy(tab_hbm.at[idx_v.at[i * _SUB + j]],
                             rawb[b].at[pl.ds(j * _G, _G)], sgb[b])

    def drain(b):
        pltpu.make_async_copy(tab_hbm.at[pl.ds(0, _CHUNK)], rawb[b], sgb[b])\
            .wait()

    def compact(b):
        raw, pk = rawb[b], pkb[b]

        def row(rr, c):
            for w in _WOFF:
                pk[rr, pl.ds(w, 16)] = raw[rr, pl.ds(w, 16)]
            return c

        lax.fori_loop(0, _CHUNK, row, 0)

    pltpu.sync_copy(x_hbm.at[pl.ds(wid * _XROWS, _XROWS)], idx_v)
    fire(0, 0)

    def step(it, c):
        for b in (0, 1):
            i = 2 * it + b
            drain(b)

            @pl.when(jnp.logical_or(b == 0, it < _STEPS // 2 - 1))
            def _():
                fire(i + 1, 1 - b)

            compact(b)
            pltpu.sync_copy(pkb[b],
                            out_hbm.at[pl.ds(obase + i * _CHUNK, _CHUNK)])
        return c

    lax.fori_loop(0, _STEPS // 2, step, 0)


def kernel(x, emb_table):
    idx = x.reshape(_B // _G, _G).astype(jnp.int32)
    tab128 = jnp.pad(emb_table, ((0, 0), (0, DP - EMBED_DIM)))

    gather = pl.kernel(
        _gather_body,
        out_type=jax.ShapeDtypeStruct((_B, EMBED_DIM), jnp.float32),
        mesh=plsc.VectorSubcoreMesh(core_axis_name="c", subcore_axis_name="s"),
        scratch_types=[
            pltpu.VMEM((_XROWS, _G), jnp.int32),
            pltpu.VMEM((_CHUNK, DP), jnp.float32),
            pltpu.VMEM((_CHUNK, DP), jnp.float32),
            pltpu.VMEM((_CHUNK, EMBED_DIM), jnp.float32),
            pltpu.VMEM((_CHUNK, EMBED_DIM), jnp.float32),
            pltpu.SemaphoreType.DMA,
            pltpu.SemaphoreType.DMA,
        ],
        compiler_params=pltpu.CompilerParams(use_tc_tiling_on_sc=True,
                                             needs_layout_passes=False),
    )
    out = gather(idx, tab128)
    return out.reshape(BATCH, SEQ, EMBED_DIM)


# speedup vs baseline: 2.1380x; 1.0027x over previous
import jax
import jax.numpy as jnp
from jax import lax
from jax.experimental import pallas as pl
from jax.experimental.pallas import tpu as pltpu
from jax.experimental.pallas import tpu_sc as plsc

VOCAB = 400000
EMBED_DIM = 50
DP = 128
BATCH = 4096
SEQ = 200

_INFO = plsc.get_sparse_core_info()
_NC = _INFO.num_cores
_NS = _INFO.num_subcores
_NW = _NC * _NS

_B = BATCH * SEQ
_PER_W = _B // _NW
_G = 128
_SUB = 1
_CHUNK = _G * _SUB
_STEPS = _PER_W // _CHUNK
_XROWS = _PER_W // _G

_WOFF = (0, 16, 32, 34)


def _gather_body(x_hbm, tab_hbm, out_hbm, idx_v, raw0, raw1, pk0, pk1,
                 sg0, sg1, ss0, ss1):
    wid = lax.axis_index("s") * _NC + lax.axis_index("c")
    obase = wid * _PER_W
    rawb, pkb = (raw0, raw1), (pk0, pk1)
    sgb, ssb = (sg0, sg1), (ss0, ss1)

    def fire(i, b):
        for j in range(_SUB):
            pltpu.async_copy(tab_hbm.at[idx_v.at[i * _SUB + j]],
                             rawb[b].at[pl.ds(j * _G, _G)], sgb[b])

    def drain(b):
        pltpu.make_async_copy(tab_hbm.at[pl.ds(0, _CHUNK)], rawb[b], sgb[b])\
            .wait()

    def compact(b):
        raw, pk = rawb[b], pkb[b]

        def rows4(rr, c):
            for u in range(4):
                r = rr * 4 + u
                for w in _WOFF:
                    pk[r, pl.ds(w, 16)] = raw[r, pl.ds(w, 16)]
            return c

        lax.fori_loop(0, _CHUNK // 4, rows4, 0)

    def store(i, b):
        pltpu.async_copy(pkb[b],
                         out_hbm.at[pl.ds(obase + i * _CHUNK, _CHUNK)],
                         ssb[b])

    def drain_store(b):
        pltpu.make_async_copy(pkb[b], out_hbm.at[pl.ds(0, _CHUNK)], ssb[b])\
            .wait()

    pltpu.sync_copy(x_hbm.at[pl.ds(wid * _XROWS, _XROWS)], idx_v)
    fire(0, 0)

    def step(it, c):
        for b in (0, 1):
            i = 2 * it + b
            drain(b)

            @pl.when(jnp.logical_or(b == 0, it < _STEPS // 2 - 1))
            def _():
                fire(i + 1, 1 - b)

            @pl.when(it > 0)
            def _():
                drain_store(b)

            compact(b)
            store(i, b)
        return c

    lax.fori_loop(0, _STEPS // 2, step, 0)
    drain_store(0)
    drain_store(1)


def kernel(x, emb_table):
    idx = x.reshape(_B // _G, _G).astype(jnp.int32)
    tab128 = jnp.pad(emb_table, ((0, 0), (0, DP - EMBED_DIM)))

    gather = pl.kernel(
        _gather_body,
        out_type=jax.ShapeDtypeStruct((_B, EMBED_DIM), jnp.float32),
        mesh=plsc.VectorSubcoreMesh(core_axis_name="c", subcore_axis_name="s"),
        scratch_types=[
            pltpu.VMEM((_XROWS, _G), jnp.int32),
            pltpu.VMEM((_CHUNK, DP), jnp.float32),
            pltpu.VMEM((_CHUNK, DP), jnp.float32),
            pltpu.VMEM((_CHUNK, EMBED_DIM), jnp.float32),
            pltpu.VMEM((_CHUNK, EMBED_DIM), jnp.float32),
            pltpu.SemaphoreType.DMA,
            pltpu.SemaphoreType.DMA,
            pltpu.SemaphoreType.DMA,
            pltpu.SemaphoreType.DMA,
        ],
        compiler_params=pltpu.CompilerParams(use_tc_tiling_on_sc=True,
                                             needs_layout_passes=False),
    )
    out = gather(idx, tab128)
    return out.reshape(BATCH, SEQ, EMBED_DIM)


# speedup vs baseline: 2.2764x; 1.0647x over previous
import jax
import jax.numpy as jnp
from jax import lax
from jax.experimental import pallas as pl
from jax.experimental.pallas import tpu as pltpu
from jax.experimental.pallas import tpu_sc as plsc

VOCAB = 400000
EMBED_DIM = 50
DP = 128
BATCH = 4096
SEQ = 200

_INFO = plsc.get_sparse_core_info()
_NC = _INFO.num_cores
_NS = _INFO.num_subcores
_NW = _NC * _NS

_B = BATCH * SEQ
_PER_W = _B // _NW
_G = 128
_SUB = 1
_CHUNK = _G * _SUB
_STEPS = _PER_W // _CHUNK
_XROWS = _PER_W // _G

_WOFF = (0, 16, 32, 34)


def _gather_body(x_hbm, tab_hbm, out_hbm, idx_v, raw0, raw1, raw2,
                 pk0, pk1, pk2, sg0, sg1, sg2, ss0, ss1, ss2):
    wid = lax.axis_index("s") * _NC + lax.axis_index("c")
    obase = wid * _PER_W
    rawb, pkb = (raw0, raw1, raw2), (pk0, pk1, pk2)
    sgb, ssb = (sg0, sg1, sg2), (ss0, ss1, ss2)

    def fire(i, b):
        for j in range(_SUB):
            pltpu.async_copy(tab_hbm.at[idx_v.at[i * _SUB + j]],
                             rawb[b].at[pl.ds(j * _G, _G)], sgb[b])

    def drain(b):
        pltpu.make_async_copy(tab_hbm.at[pl.ds(0, _CHUNK)], rawb[b], sgb[b])\
            .wait()

    def compact(b):
        raw, pk = rawb[b], pkb[b]

        def rows4(rr, c):
            for u in range(4):
                r = rr * 4 + u
                for w in _WOFF:
                    pk[r, pl.ds(w, 16)] = raw[r, pl.ds(w, 16)]
            return c

        lax.fori_loop(0, _CHUNK // 4, rows4, 0)

    def store(i, b):
        pltpu.async_copy(pkb[b],
                         out_hbm.at[pl.ds(obase + i * _CHUNK, _CHUNK)],
                         ssb[b])

    def drain_store(b):
        pltpu.make_async_copy(pkb[b], out_hbm.at[pl.ds(0, _CHUNK)], ssb[b])\
            .wait()

    pltpu.sync_copy(x_hbm.at[pl.ds(wid * _XROWS, _XROWS)], idx_v)
    fire(0, 0)
    fire(1, 1)

    def step(it, c):
        for b in (0, 1, 2):
            i = 3 * it + b
            drain(b)
            fire(i + 2, (b + 2) % 3)

            @pl.when(it > 0)
            def _():
                drain_store(b)

            compact(b)
            store(i, b)
        return c

    lax.fori_loop(0, (_STEPS - 2) // 3, step, 0)

    for i in (_STEPS - 2, _STEPS - 1):
        b = i % 3
        drain(b)
        drain_store(b)
        compact(b)
        store(i, b)
    for b in (0, 1, 2):
        drain_store(b)


def kernel(x, emb_table):
    idx = x.reshape(_B // _G, _G).astype(jnp.int32)
    tab128 = jnp.pad(emb_table, ((0, 0), (0, DP - EMBED_DIM)))

    gather = pl.kernel(
        _gather_body,
        out_type=jax.ShapeDtypeStruct((_B, EMBED_DIM), jnp.float32),
        mesh=plsc.VectorSubcoreMesh(core_axis_name="c", subcore_axis_name="s"),
        scratch_types=[
            pltpu.VMEM((_XROWS, _G), jnp.int32),
            pltpu.VMEM((_CHUNK, DP), jnp.float32),
            pltpu.VMEM((_CHUNK, DP), jnp.float32),
            pltpu.VMEM((_CHUNK, DP), jnp.float32),
            pltpu.VMEM((_CHUNK, EMBED_DIM), jnp.float32),
            pltpu.VMEM((_CHUNK, EMBED_DIM), jnp.float32),
            pltpu.VMEM((_CHUNK, EMBED_DIM), jnp.float32),
            pltpu.SemaphoreType.DMA,
            pltpu.SemaphoreType.DMA,
            pltpu.SemaphoreType.DMA,
            pltpu.SemaphoreType.DMA,
            pltpu.SemaphoreType.DMA,
            pltpu.SemaphoreType.DMA,
        ],
        compiler_params=pltpu.CompilerParams(use_tc_tiling_on_sc=True,
                                             needs_layout_passes=False),
    )
    out = gather(idx, tab128)
    return out.reshape(BATCH, SEQ, EMBED_DIM)
